# baseline (speedup 1.0000x reference)
import jax
import jax.numpy as jnp
from jax import lax
from jax.experimental import pallas as pl
from jax.experimental.pallas import tpu as pltpu
from jax.experimental.pallas import tpu_sc as plsc

_B, _Y, _X = 16, 4096, 512
_NEG = -10000
_L = 16
_UNROLL = 8


def _body(dur_hbm, olen_hbm, ilen_hbm, wa_hbm, d_hbm,
          dbuf, wbuf, obuf, lbuf, ibuf, sem):
    w = lax.axis_index("s")

    @pl.when(w < _B)
    def _():
        row = w
        in_cp = pltpu.async_copy(dur_hbm.at[row], dbuf.at[pl.ds(0, _Y)], sem)
        pltpu.sync_copy(olen_hbm, lbuf)
        pltpu.sync_copy(ilen_hbm, ibuf)

        lane = lax.iota(jnp.int32, _L)
        lvec = lbuf[...]
        ivec = ibuf[...]
        out_len = jnp.max(jnp.where(lane == row, lvec, 0))
        max_in = jnp.max(ivec)

        zeros = jnp.zeros((_L,), jnp.int32)

        def zero_o(j, carry):
            obuf[pl.ds(j * _L, _L)] = zeros
            return carry

        lax.fori_loop(0, _X // _L, zero_o, 0)
        in_cp.wait()

        def pass_row(i, carry):
            for u in range(_UNROLL):
                base = (i * _UNROLL + u) * _L
                pos = base + lane
                val = dbuf[pl.ds(base, _L)]
                wbuf[pl.ds(base, _L)] = jnp.where(pos < out_len, val,
                                                  val + _NEG)
                nxt = plsc.load_gather(dbuf, [pos + 1])
                is_last = (val != nxt) | (pos == _Y - 1)
                m1 = jnp.minimum(pos + 1, out_len)
                plsc.addupdate_scatter(
                    obuf, [val], m1, mask=is_last & (val < max_in))
                plsc.addupdate_scatter(
                    obuf, [nxt], -m1,
                    mask=is_last & (pos < _Y - 1) & (nxt < max_in))
            return carry

        lax.fori_loop(0, _Y // (_L * _UNROLL), pass_row, 0)

        pltpu.sync_copy(wbuf, wa_hbm.at[row])
        pltpu.sync_copy(obuf, d_hbm.at[row])


@jax.jit
def kernel(duration, output_length, input_length):
    mesh = plsc.VectorSubcoreMesh(
        core_axis_name="c", subcore_axis_name="s", num_cores=1)
    run = pl.kernel(
        _body,
        out_type=(
            jax.ShapeDtypeStruct((_B, _Y), jnp.int32),
            jax.ShapeDtypeStruct((_B, _X), jnp.int32),
        ),
        mesh=mesh,
        compiler_params=pltpu.CompilerParams(needs_layout_passes=False),
        scratch_types=[
            pltpu.VMEM((_Y + _L,), jnp.int32),
            pltpu.VMEM((_Y,), jnp.int32),
            pltpu.VMEM((_X,), jnp.int32),
            pltpu.VMEM((_L,), jnp.int32),
            pltpu.VMEM((_L,), jnp.int32),
            pltpu.SemaphoreType.DMA,
        ],
    )
    return run(duration, output_length, input_length)


# speedup vs baseline: 1.0503x; 1.0503x over previous
import jax
import jax.numpy as jnp
from jax import lax
from jax.experimental import pallas as pl
from jax.experimental.pallas import tpu as pltpu
from jax.experimental.pallas import tpu_sc as plsc

_B, _Y, _X = 16, 4096, 512
_NEG = -10000
_L = 16
_UNROLL = 8


def _sc_hist(dur_hbm, olen_hbm, ilen_hbm, d_hbm, dbuf, obuf, lbuf, ibuf, sem):
    w = lax.axis_index("s")

    @pl.when(w < _B)
    def _():
        row = w
        in_cp = pltpu.async_copy(dur_hbm.at[row], dbuf.at[pl.ds(0, _Y)], sem)
        pltpu.sync_copy(olen_hbm, lbuf)
        pltpu.sync_copy(ilen_hbm, ibuf)

        lane = lax.iota(jnp.int32, _L)
        out_len = jnp.max(jnp.where(lane == row, lbuf[...], 0))
        max_in = jnp.max(ibuf[...])

        zeros = jnp.zeros((_L,), jnp.int32)

        def zero_o(j, carry):
            obuf[pl.ds(j * _L, _L)] = zeros
            return carry

        lax.fori_loop(0, _X // _L, zero_o, 0)
        in_cp.wait()

        def pass_row(i, carry):
            for u in range(_UNROLL):
                base = (i * _UNROLL + u) * _L
                pos = base + lane
                val = dbuf[pl.ds(base, _L)]
                nxt = plsc.load_gather(dbuf, [pos + 1])
                is_last = (val != nxt) | (pos == _Y - 1)
                m1 = jnp.minimum(pos + 1, out_len)
                plsc.addupdate_scatter(
                    obuf, [val], m1, mask=is_last & (val < max_in))
                plsc.addupdate_scatter(
                    obuf, [nxt], -m1,
                    mask=is_last & (pos < _Y - 1) & (nxt < max_in))
            return carry

        lax.fori_loop(0, _Y // (_L * _UNROLL), pass_row, 0)

        pltpu.sync_copy(obuf, d_hbm.at[row])


def _tc_weights(dur_ref, olen_ref, out_ref):
    pos = lax.broadcasted_iota(jnp.int32, (_B, _Y), 1)
    mask = pos < olen_ref[...]
    dur = dur_ref[...]
    out_ref[...] = jnp.where(mask, dur, dur + _NEG)


@jax.jit
def kernel(duration, output_length, input_length):
    mesh = plsc.VectorSubcoreMesh(
        core_axis_name="c", subcore_axis_name="s", num_cores=1)
    hist = pl.kernel(
        _sc_hist,
        out_type=jax.ShapeDtypeStruct((_B, _X), jnp.int32),
        mesh=mesh,
        compiler_params=pltpu.CompilerParams(needs_layout_passes=False),
        scratch_types=[
            pltpu.VMEM((_Y + _L,), jnp.int32),
            pltpu.VMEM((_X,), jnp.int32),
            pltpu.VMEM((_L,), jnp.int32),
            pltpu.VMEM((_L,), jnp.int32),
            pltpu.SemaphoreType.DMA,
        ],
    )
    durations = hist(duration, output_length, input_length)

    weights = pl.pallas_call(
        _tc_weights,
        out_shape=jax.ShapeDtypeStruct((_B, _Y), jnp.int32),
    )(duration, output_length.reshape(_B, 1))

    return (weights, durations)
